# baseline (speedup 1.0000x reference)
import jax
import jax.numpy as jnp
from jax.experimental import pallas as pl
from jax.experimental.pallas import tpu as pltpu

_PAD = 6
_K = 25
_INV_K = 1.0 / _K


def _mavg_kernel(x_ref, o_ref):
    x = x_ref[...]
    bb, L, C = x.shape
    M = o_ref.shape[1]
    z = jnp.zeros((bb, 8, C), x.dtype)
    xp = jnp.concatenate([z, x, z], axis=1)
    s8 = xp[:, 0:L] + xp[:, 8:L + 8] + xp[:, 16:L + 16]
    p2 = s8[:, 0:L - 1] + s8[:, 1:L]
    p4 = p2[:, 0:L - 3] + p2[:, 2:L - 1]
    p8 = p4[:, 0:L - 7] + p4[:, 4:L - 3]
    w = p8[:, 3:M + 3] + xp[:, 27:M + 27]
    o_ref[...] = w * jnp.float32(_INV_K)


def kernel(x):
    B, L, C = x.shape
    L_pool = (L + 2 * _PAD - _K) // 1 + 1
    M = L_pool - 1

    bb = 4
    while B % bb:
        bb //= 2
    grid = (B // bb,)

    return pl.pallas_call(
        _mavg_kernel,
        out_shape=jax.ShapeDtypeStruct((B, M, C), x.dtype),
        grid=grid,
        in_specs=[pl.BlockSpec((bb, L, C), lambda i: (i, 0, 0))],
        out_specs=pl.BlockSpec((bb, M, C), lambda i: (i, 0, 0)),
        compiler_params=pltpu.CompilerParams(
            dimension_semantics=("parallel",),
            vmem_limit_bytes=64 * 1024 * 1024),
    )(x)


# speedup vs baseline: 1.0691x; 1.0691x over previous
import jax
import jax.numpy as jnp
from jax.experimental import pallas as pl
from jax.experimental.pallas import tpu as pltpu

_PAD = 6
_K = 25
_INV_K = 1.0 / _K


def _mavg_kernel(x_ref, o_ref):
    x = x_ref[...]
    bb, L, C = x.shape
    M = o_ref.shape[1]
    z = jnp.zeros((bb, 8, C), x.dtype)
    xp = jnp.concatenate([z, x, z], axis=1)
    s8 = xp[:, 0:L] + xp[:, 8:L + 8] + xp[:, 16:L + 16]
    p2 = s8[:, 0:L - 1] + s8[:, 1:L]
    p4 = p2[:, 0:L - 3] + p2[:, 2:L - 1]
    p8 = p4[:, 0:L - 7] + p4[:, 4:L - 3]
    w = p8[:, 3:M + 3] + xp[:, 27:M + 27]
    o_ref[...] = w * jnp.float32(_INV_K)


def kernel(x):
    B, L, C = x.shape
    L_pool = (L + 2 * _PAD - _K) // 1 + 1
    M = L_pool - 1

    bb = 8
    while B % bb:
        bb //= 2
    grid = (B // bb,)

    return pl.pallas_call(
        _mavg_kernel,
        out_shape=jax.ShapeDtypeStruct((B, M, C), x.dtype),
        grid=grid,
        in_specs=[pl.BlockSpec((bb, L, C), lambda i: (i, 0, 0))],
        out_specs=pl.BlockSpec((bb, M, C), lambda i: (i, 0, 0)),
        compiler_params=pltpu.CompilerParams(
            dimension_semantics=("parallel",),
            vmem_limit_bytes=64 * 1024 * 1024),
    )(x)


# speedup vs baseline: 1.1547x; 1.0800x over previous
import functools

import jax
import jax.numpy as jnp
from jax.experimental import pallas as pl
from jax.experimental.pallas import tpu as pltpu

_PAD = 6
_K = 25
_INV_K = 1.0 / _K
_BB = 4


def _window_sum(x, M):
    bb, Lp, C = x.shape
    L = Lp - 16
    s8 = x[:, 0:L] + x[:, 8:L + 8] + x[:, 16:L + 16]
    p2 = s8[:, 0:L - 1] + s8[:, 1:L]
    p4 = p2[:, 0:L - 3] + p2[:, 2:L - 1]
    p8 = p4[:, 0:L - 7] + p4[:, 4:L - 3]
    return p8[:, 3:M + 3] + x[:, 27:M + 27]


def _mavg_kernel(x_hbm, o_hbm, xbuf, obuf, in_sems, out_sems, *, nb, nchunks):
    core = pl.program_id(0)
    base = core * nb
    bb = _BB
    M = o_hbm.shape[1]
    L = x_hbm.shape[1]

    xbuf[:, 0:8] = jnp.zeros((nb, 8, xbuf.shape[2]), xbuf.dtype)
    xbuf[:, L + 8:L + 16] = jnp.zeros((nb, 8, xbuf.shape[2]), xbuf.dtype)

    for k in range(nchunks):
        pltpu.make_async_copy(
            x_hbm.at[pl.ds(base + k * bb, bb)],
            xbuf.at[pl.ds(k * bb, bb), pl.ds(8, x_hbm.shape[1])],
            in_sems.at[k]).start()

    for k in range(nchunks):
        pltpu.make_async_copy(
            x_hbm.at[pl.ds(base + k * bb, bb)],
            xbuf.at[pl.ds(k * bb, bb), pl.ds(8, x_hbm.shape[1])],
            in_sems.at[k]).wait()
        xc = xbuf[pl.ds(k * bb, bb)]
        obuf[pl.ds(k * bb, bb)] = _window_sum(xc, M) * jnp.float32(_INV_K)
        pltpu.make_async_copy(
            obuf.at[pl.ds(k * bb, bb)],
            o_hbm.at[pl.ds(base + k * bb, bb)],
            out_sems.at[k]).start()

    for k in range(nchunks):
        pltpu.make_async_copy(
            obuf.at[pl.ds(k * bb, bb)],
            o_hbm.at[pl.ds(base + k * bb, bb)],
            out_sems.at[k]).wait()


def kernel(x):
    B, L, C = x.shape
    L_pool = (L + 2 * _PAD - _K) // 1 + 1
    M = L_pool - 1

    nb = B // 2
    nchunks = nb // _BB

    kfn = functools.partial(_mavg_kernel, nb=nb, nchunks=nchunks)

    return pl.pallas_call(
        kfn,
        out_shape=jax.ShapeDtypeStruct((B, M, C), x.dtype),
        grid=(2,),
        in_specs=[pl.BlockSpec(memory_space=pl.ANY)],
        out_specs=pl.BlockSpec(memory_space=pl.ANY),
        scratch_shapes=[
            pltpu.VMEM((nb, L + 16, C), x.dtype),
            pltpu.VMEM((nb, M, C), x.dtype),
            pltpu.SemaphoreType.DMA((nchunks,)),
            pltpu.SemaphoreType.DMA((nchunks,)),
        ],
        compiler_params=pltpu.CompilerParams(
            dimension_semantics=("parallel",),
            vmem_limit_bytes=100 * 1024 * 1024),
    )(x)


# speedup vs baseline: 1.1704x; 1.0136x over previous
import functools

import jax
import jax.numpy as jnp
from jax.experimental import pallas as pl
from jax.experimental.pallas import tpu as pltpu

_PAD = 6
_K = 25
_INV_K = 1.0 / _K
_BB = 2


def _window_sum(x, M):
    bb, Lp, C = x.shape
    L = Lp - 16
    s8 = x[:, 0:L] + x[:, 8:L + 8] + x[:, 16:L + 16]
    p2 = s8[:, 0:L - 1] + s8[:, 1:L]
    p4 = p2[:, 0:L - 3] + p2[:, 2:L - 1]
    p8 = p4[:, 0:L - 7] + p4[:, 4:L - 3]
    return p8[:, 3:M + 3] + x[:, 27:M + 27]


def _mavg_kernel(x_hbm, o_hbm, xbuf, obuf, in_sems, out_sems, *, nb, nchunks):
    core = pl.program_id(0)
    base = core * nb
    bb = _BB
    M = o_hbm.shape[1]
    L = x_hbm.shape[1]

    xbuf[:, 0:8] = jnp.zeros((nb, 8, xbuf.shape[2]), xbuf.dtype)
    xbuf[:, L + 8:L + 16] = jnp.zeros((nb, 8, xbuf.shape[2]), xbuf.dtype)

    for k in range(nchunks):
        pltpu.make_async_copy(
            x_hbm.at[pl.ds(base + k * bb, bb)],
            xbuf.at[pl.ds(k * bb, bb), pl.ds(8, x_hbm.shape[1])],
            in_sems.at[k]).start()

    for k in range(nchunks):
        pltpu.make_async_copy(
            x_hbm.at[pl.ds(base + k * bb, bb)],
            xbuf.at[pl.ds(k * bb, bb), pl.ds(8, x_hbm.shape[1])],
            in_sems.at[k]).wait()
        xc = xbuf[pl.ds(k * bb, bb)]
        obuf[pl.ds(k * bb, bb)] = _window_sum(xc, M) * jnp.float32(_INV_K)
        pltpu.make_async_copy(
            obuf.at[pl.ds(k * bb, bb)],
            o_hbm.at[pl.ds(base + k * bb, bb)],
            out_sems.at[k]).start()

    for k in range(nchunks):
        pltpu.make_async_copy(
            obuf.at[pl.ds(k * bb, bb)],
            o_hbm.at[pl.ds(base + k * bb, bb)],
            out_sems.at[k]).wait()


def kernel(x):
    B, L, C = x.shape
    L_pool = (L + 2 * _PAD - _K) // 1 + 1
    M = L_pool - 1

    nb = B // 2
    nchunks = nb // _BB

    kfn = functools.partial(_mavg_kernel, nb=nb, nchunks=nchunks)

    return pl.pallas_call(
        kfn,
        out_shape=jax.ShapeDtypeStruct((B, M, C), x.dtype),
        grid=(2,),
        in_specs=[pl.BlockSpec(memory_space=pl.ANY)],
        out_specs=pl.BlockSpec(memory_space=pl.ANY),
        scratch_shapes=[
            pltpu.VMEM((nb, L + 16, C), x.dtype),
            pltpu.VMEM((nb, M, C), x.dtype),
            pltpu.SemaphoreType.DMA((nchunks,)),
            pltpu.SemaphoreType.DMA((nchunks,)),
        ],
        compiler_params=pltpu.CompilerParams(
            dimension_semantics=("parallel",),
            vmem_limit_bytes=100 * 1024 * 1024),
    )(x)
